# baseline (speedup 1.0000x reference)
import functools

import numpy as np

import jax
import jax.numpy as jnp
from jax import lax
from jax.experimental import pallas as pl
from jax.experimental.pallas import tpu as pltpu
from jax.experimental.pallas import tpu_sc as plsc

_N, _C, _H, _W = 128, 1000, 14, 14
_CAMS = 100
_K = _CAMS - 1
_CT = 200
_IMIN = np.int32(-(2 ** 31))


def _keyify(x):
    x = x + 0.0
    bits = lax.bitcast_convert_type(x, jnp.int32)
    mag = bits & jnp.int32(0x7FFFFFFF)
    return jnp.where(bits >= 0, bits, -mag)


def _sc_thresh_kernel(lt_hbm, idx_hbm, out_hbm, idx_v, col_v, key_v, th_v,
                      sem):
    c = lax.axis_index("c")
    s = lax.axis_index("s")
    iota = lax.iota(jnp.int32, 16)

    tvec = jnp.zeros((16,), jnp.int32)
    for k in range(4):
        n = 64 * c + 4 * s + k
        pltpu.sync_copy(idx_hbm.at[n], idx_v)
        for j in range(8):
            pltpu.async_copy(
                lt_hbm.at[idx_v.at[j]],
                col_v.at[j],
                sem,
            ).wait()
        for j in range(8):
            sl = pl.ds(112, 16)
            col_v[j, sl] = jnp.where(iota < 13, col_v[j, sl],
                                     jnp.float32(float("-inf")))
        for j in range(8):
            for v in range(8):
                sl = pl.ds(16 * v, 16)
                key_v[j, sl] = _keyify(col_v[j, sl])

        def body(i, u):
            bit = jnp.int32(1) << (jnp.int32(31) - i)
            u_c = u | bit
            v_c = u_c ^ _IMIN
            cnt = jnp.zeros((16,), jnp.int32)
            for j in range(8):
                for v in range(8):
                    kv = key_v[j, pl.ds(16 * v, 16)]
                    cnt = cnt + jnp.where(kv >= v_c, jnp.int32(1),
                                          jnp.int32(0))
            total = cnt[0]
            for l in range(1, 16):
                total = total + cnt[l]
            return jnp.where(total >= _CAMS, u_c, u)

        u = lax.fori_loop(0, 32, body, jnp.int32(0))
        thr = u ^ _IMIN
        tvec = jnp.where(iota == k, thr, tvec)

    th_v[...] = tvec
    pltpu.sync_copy(th_v, out_hbm.at[s * 2 + c])


def _sc_thresholds(lt_flat, idx_tab):
    mesh = plsc.VectorSubcoreMesh(core_axis_name="c", subcore_axis_name="s")
    kfn = functools.partial(
        pl.kernel,
        mesh=mesh,
        out_type=jax.ShapeDtypeStruct((32, 16), jnp.int32),
        scratch_types=[
            pltpu.VMEM((8, 128), jnp.int32),
            pltpu.VMEM((8, 128), jnp.float32),
            pltpu.VMEM((8, 128), jnp.int32),
            pltpu.VMEM((16,), jnp.int32),
            pltpu.SemaphoreType.DMA,
        ],
    )(_sc_thresh_kernel)
    t = kfn(lt_flat, idx_tab)
    n = jnp.arange(_N, dtype=jnp.int32)
    w = 2 * ((n % 64) // 4) + n // 64
    return t[w, n % 4]


def _make_idx_tab():
    r = jnp.minimum(125 * jnp.arange(8, dtype=jnp.int32)[:, None]
                    + jnp.arange(128, dtype=jnp.int32)[None, :],
                    _C - 1)
    return (jnp.arange(_N, dtype=jnp.int32)[:, None, None]
            + _N * r[None, :, :])


def _loss_kernel(thr_ref, lt_ref, fs_ref, ft_ref, out_ref, mask_ref):
    ci = pl.program_id(0)
    qh = pl.program_id(1)
    qw = pl.program_id(2)

    @pl.when((ci == 0) & (qh == 0) & (qw == 0))
    def _():
        key = _keyify(lt_ref[...])
        mask_ref[...] = (key > thr_ref[...]).astype(jnp.float32)
        out_ref[0, 0] = 0.0

    d = fs_ref[...] - ft_ref[...]
    s = jnp.sum(d, axis=(0, 1))
    w = mask_ref[pl.ds(ci * _CT, _CT), :]
    out_ref[0, 0] += jnp.sum(s * s * w)


def kernel(f_s, f_t, l_t):
    fs_t = jnp.transpose(f_s, (2, 3, 1, 0))
    ft_t = jnp.transpose(f_t, (2, 3, 1, 0))
    lt_t = jnp.transpose(l_t, (1, 0))
    thr = _sc_thresholds(lt_t.reshape(-1), _make_idx_tab()).reshape(1, _N)
    acc = pl.pallas_call(
        _loss_kernel,
        grid=(_C // _CT, 2, 2),
        in_specs=[
            pl.BlockSpec((1, _N), lambda ci, qh, qw: (0, 0)),
            pl.BlockSpec((_C, _N), lambda ci, qh, qw: (0, 0)),
            pl.BlockSpec((7, 7, _CT, _N), lambda ci, qh, qw: (qh, qw, ci, 0)),
            pl.BlockSpec((7, 7, _CT, _N), lambda ci, qh, qw: (qh, qw, ci, 0)),
        ],
        out_specs=pl.BlockSpec(
            (1, 1), lambda ci, qh, qw: (0, 0), memory_space=pltpu.SMEM
        ),
        out_shape=jax.ShapeDtypeStruct((1, 1), jnp.float32),
        scratch_shapes=[pltpu.VMEM((_C, _N), jnp.float32)],
    )(thr, lt_t, fs_t, ft_t)
    scale = 1.0 / (49.0 * 49.0 * _N * _K * 4.0)
    return (acc[0, 0] * scale).astype(jnp.float32)


# speedup vs baseline: 1.3281x; 1.3281x over previous
import functools

import numpy as np

import jax
import jax.numpy as jnp
from jax import lax
from jax.experimental import pallas as pl
from jax.experimental.pallas import tpu as pltpu
from jax.experimental.pallas import tpu_sc as plsc

_N, _C, _H, _W = 128, 1000, 14, 14
_CAMS = 100
_K = _CAMS - 1
_CT = 200
_IMIN = np.int32(-(2 ** 31))


def _keyify(x):
    x = x + 0.0
    bits = lax.bitcast_convert_type(x, jnp.int32)
    mag = bits & jnp.int32(0x7FFFFFFF)
    return jnp.where(bits >= 0, bits, -mag)


def _sc_thresh_kernel(lt_hbm, idx_hbm, out_hbm, idx_v, col_v, key_v, th_v,
                      sem):
    c = lax.axis_index("c")
    s = lax.axis_index("s")
    iota = lax.iota(jnp.int32, 16)

    tvec = jnp.zeros((16,), jnp.int32)
    for k in range(4):
        n = 64 * c + 4 * s + k
        pltpu.sync_copy(idx_hbm.at[n], idx_v)
        for j in range(8):
            pltpu.async_copy(
                lt_hbm.at[idx_v.at[j]],
                col_v.at[j],
                sem,
            ).wait()
        for j in range(8):
            sl = pl.ds(112, 16)
            col_v[j, sl] = jnp.where(iota < 13, col_v[j, sl],
                                     jnp.float32(float("-inf")))
        for j in range(8):
            for v in range(8):
                sl = pl.ds(16 * v, 16)
                key_v[j, sl] = _keyify(col_v[j, sl])

        def body(i, u):
            bit = jnp.int32(1) << (jnp.int32(31) - i)
            u_c = u | bit
            v_c = u_c ^ _IMIN
            cnt = jnp.zeros((16,), jnp.int32)
            for j in range(8):
                for v in range(8):
                    kv = key_v[j, pl.ds(16 * v, 16)]
                    cnt = cnt + jnp.where(kv >= v_c, jnp.int32(1),
                                          jnp.int32(0))
            total = cnt[0]
            for l in range(1, 16):
                total = total + cnt[l]
            return jnp.where(total >= _CAMS, u_c, u)

        u = lax.fori_loop(0, 32, body, jnp.int32(0))
        thr = u ^ _IMIN
        tvec = jnp.where(iota == k, thr, tvec)

    th_v[...] = tvec
    pltpu.sync_copy(th_v, out_hbm.at[s * 2 + c])


def _sc_thresholds(lt_flat, idx_tab):
    mesh = plsc.VectorSubcoreMesh(core_axis_name="c", subcore_axis_name="s")
    kfn = functools.partial(
        pl.kernel,
        mesh=mesh,
        out_type=jax.ShapeDtypeStruct((32, 16), jnp.int32),
        scratch_types=[
            pltpu.VMEM((8, 128), jnp.int32),
            pltpu.VMEM((8, 128), jnp.float32),
            pltpu.VMEM((8, 128), jnp.int32),
            pltpu.VMEM((16,), jnp.int32),
            pltpu.SemaphoreType.DMA,
        ],
    )(_sc_thresh_kernel)
    t = kfn(lt_flat, idx_tab)
    n = jnp.arange(_N, dtype=jnp.int32)
    w = 2 * ((n % 64) // 4) + n // 64
    return t[w, n % 4]


def _make_idx_tab():
    r = jnp.minimum(125 * jnp.arange(8, dtype=jnp.int32)[:, None]
                    + jnp.arange(128, dtype=jnp.int32)[None, :],
                    _C - 1)
    return (jnp.arange(_N, dtype=jnp.int32)[:, None, None]
            + _N * r[None, :, :])


def _energy_kernel(fs_ref, ft_ref, e_ref):
    qh = pl.program_id(1)
    qw = pl.program_id(2)

    @pl.when((qh == 0) & (qw == 0))
    def _():
        e_ref[...] = jnp.zeros_like(e_ref)

    d = fs_ref[...] - ft_ref[...]
    s = jnp.sum(d, axis=(0, 1))
    e_ref[...] += s * s


def _combine_kernel(thr_ref, lt_ref, e_ref, out_ref):
    key = _keyify(lt_ref[...])
    w = (key > thr_ref[...]).astype(jnp.float32)
    out_ref[0, 0] = jnp.sum(e_ref[...] * w)


def kernel(f_s, f_t, l_t):
    fs_t = jnp.transpose(f_s, (2, 3, 1, 0))
    ft_t = jnp.transpose(f_t, (2, 3, 1, 0))
    lt_t = jnp.transpose(l_t, (1, 0))
    thr = _sc_thresholds(lt_t.reshape(-1), _make_idx_tab()).reshape(1, _N)
    e = pl.pallas_call(
        _energy_kernel,
        grid=(_C // _CT, 2, 2),
        in_specs=[
            pl.BlockSpec((7, 7, _CT, _N), lambda ci, qh, qw: (qh, qw, ci, 0)),
            pl.BlockSpec((7, 7, _CT, _N), lambda ci, qh, qw: (qh, qw, ci, 0)),
        ],
        out_specs=pl.BlockSpec((_CT, _N), lambda ci, qh, qw: (ci, 0)),
        out_shape=jax.ShapeDtypeStruct((_C, _N), jnp.float32),
    )(fs_t, ft_t)
    acc = pl.pallas_call(
        _combine_kernel,
        in_specs=[
            pl.BlockSpec((1, _N), lambda: (0, 0)),
            pl.BlockSpec((_C, _N), lambda: (0, 0)),
            pl.BlockSpec((_C, _N), lambda: (0, 0)),
        ],
        out_specs=pl.BlockSpec(
            (1, 1), lambda: (0, 0), memory_space=pltpu.SMEM
        ),
        out_shape=jax.ShapeDtypeStruct((1, 1), jnp.float32),
    )(thr, lt_t, e)
    scale = 1.0 / (49.0 * 49.0 * _N * _K * 4.0)
    return (acc[0, 0] * scale).astype(jnp.float32)


# speedup vs baseline: 1.8562x; 1.3976x over previous
import jax
import jax.numpy as jnp
from jax.experimental import pallas as pl
from jax.experimental.pallas import tpu as pltpu

_N, _C, _H, _W = 128, 1000, 14, 14
_CAMS = 100
_K = _CAMS - 1
_CT = 200


def _loss_kernel(lt_ref, fs_ref, ft_ref, out_ref, mask_ref):
    qh = pl.program_id(0)
    qw = pl.program_id(1)
    ci = pl.program_id(2)

    @pl.when((qh == 0) & (qw == 0) & (ci == 0))
    def _():
        x = lt_ref[...] + 0.0
        bits = jax.lax.bitcast_convert_type(x, jnp.int32)
        imin = jnp.int32(-(2**31))
        mag = bits & jnp.int32(0x7FFFFFFF)
        key = jnp.where(bits >= 0, bits, -mag)
        u = jnp.zeros((1, _N), jnp.int32)
        for b in range(31, -1, -1):
            u_c = u | (imin if b == 31 else jnp.int32(1 << b))
            v_c = u_c ^ imin
            cnt = jnp.sum((key >= v_c).astype(jnp.int32), axis=0, keepdims=True)
            u = jnp.where(cnt >= _CAMS, u_c, u)
        v = u ^ imin
        mask_ref[...] = (key > v).astype(jnp.float32)
        out_ref[0, 0] = 0.0

    d = fs_ref[...] - ft_ref[...]
    s = jnp.sum(d, axis=(0, 1))
    w = mask_ref[pl.ds(ci * _CT, _CT), :]
    out_ref[0, 0] += jnp.sum(s * s * w)


def kernel(f_s, f_t, l_t):
    fs_t = jnp.transpose(f_s, (2, 3, 1, 0))
    ft_t = jnp.transpose(f_t, (2, 3, 1, 0))
    lt_t = jnp.transpose(l_t, (1, 0))
    acc = pl.pallas_call(
        _loss_kernel,
        grid=(2, 2, _C // _CT),
        in_specs=[
            pl.BlockSpec((_C, _N), lambda qh, qw, ci: (0, 0)),
            pl.BlockSpec((7, 7, _CT, _N), lambda qh, qw, ci: (qh, qw, ci, 0)),
            pl.BlockSpec((7, 7, _CT, _N), lambda qh, qw, ci: (qh, qw, ci, 0)),
        ],
        out_specs=pl.BlockSpec(
            (1, 1), lambda qh, qw, ci: (0, 0), memory_space=pltpu.SMEM
        ),
        out_shape=jax.ShapeDtypeStruct((1, 1), jnp.float32),
        scratch_shapes=[pltpu.VMEM((_C, _N), jnp.float32)],
    )(lt_t, fs_t, ft_t)
    scale = 1.0 / (49.0 * 49.0 * _N * _K * 4.0)
    return (acc[0, 0] * scale).astype(jnp.float32)
